# baseline (speedup 1.0000x reference)
import jax
import jax.numpy as jnp
from jax import lax
from jax.experimental import pallas as pl
from jax.experimental.pallas import tpu as pltpu
from jax.experimental.pallas import tpu_sc as plsc

PRE_SEQ_LEN = 128
OUT_DIM = 49152
B = 8
PLEN = 128
NITEMS = B * PLEN
NCHUNK = 16
DC = OUT_DIM // NCHUNK
NC = 2
NS = 16
NW = NC * NS
ROWS_PER_W = NITEMS // NW
NG = ROWS_PER_W // 16


def _sc_body(prefix_hbm, table_hbm, out_hbm, pref_v, row_v, gsem, wsem):
    wid = lax.axis_index("s") * NC + lax.axis_index("c")
    base = wid * ROWS_PER_W
    pltpu.sync_copy(prefix_hbm.at[pl.ds(base, ROWS_PER_W)], pref_v)

    lanes = lax.iota(jnp.int32, 16)

    def step(c, carry):
        for h in range(NG):
            pv = pref_v[pl.ds(16 * h, 16)]
            gidx = pv * NCHUNK + c
            oidx = (base + 16 * h + lanes) * NCHUNK + c
            pltpu.async_copy(table_hbm.at[gidx], row_v.at[h], gsem).wait()
            pltpu.async_copy(row_v.at[h], out_hbm.at[oidx], wsem).wait()
        return carry

    lax.fori_loop(0, NCHUNK, step, 0)


@jax.jit
def _sc_gather(prefix_flat, table_r):
    mesh = plsc.VectorSubcoreMesh(core_axis_name="c", subcore_axis_name="s")
    f = pl.kernel(
        _sc_body,
        out_type=jax.ShapeDtypeStruct((NITEMS * NCHUNK, DC), jnp.float32),
        mesh=mesh,
        scratch_types=[
            pltpu.VMEM((ROWS_PER_W,), jnp.int32),
            pltpu.VMEM((NG, 16, DC), jnp.float32),
            pltpu.SemaphoreType.DMA,
            pltpu.SemaphoreType.DMA,
        ],
    )
    return f(prefix_flat, table_r)


def kernel(prefix, table):
    prefix_flat = prefix.reshape(NITEMS)
    table_r = table.reshape(PRE_SEQ_LEN * NCHUNK, DC)
    out_r = _sc_gather(prefix_flat, table_r)
    return out_r.reshape(B, PLEN, OUT_DIM)


# speedup vs baseline: 1.0757x; 1.0757x over previous
import jax
import jax.numpy as jnp
from jax import lax
from jax.experimental import pallas as pl
from jax.experimental.pallas import tpu as pltpu
from jax.experimental.pallas import tpu_sc as plsc

PRE_SEQ_LEN = 128
OUT_DIM = 49152
B = 8
PLEN = 128
NITEMS = B * PLEN
NCHUNK = 32
DC = OUT_DIM // NCHUNK
NC = 2
NS = 16
NW = NC * NS
ROWS_PER_W = NITEMS // NW
NG = ROWS_PER_W // 16
RING = 4
DEFER = RING // 2
NB = NCHUNK // RING


def _sc_body(prefix_hbm, table_hbm, out_hbm, pref_v, row_v, gsem, wsem):
    wid = lax.axis_index("s") * NC + lax.axis_index("c")
    base = wid * ROWS_PER_W
    pltpu.sync_copy(prefix_hbm.at[pl.ds(base, ROWS_PER_W)], pref_v)

    lanes = lax.iota(jnp.int32, 16)

    for h in range(NG):
        pv = pref_v[pl.ds(16 * h, 16)]
        oitems = (base + 16 * h + lanes) * NCHUNK

        def fire_gather(c, slot):
            pltpu.async_copy(table_hbm.at[pv * NCHUNK + c], row_v.at[slot],
                             gsem.at[slot])

        def wait_gather(c, slot):
            pltpu.make_async_copy(table_hbm.at[pv * NCHUNK + c],
                                  row_v.at[slot], gsem.at[slot]).wait()

        def fire_scatter(c, slot):
            pltpu.async_copy(row_v.at[slot], out_hbm.at[oitems + c],
                             wsem.at[slot])

        def wait_scatter(c, slot):
            pltpu.make_async_copy(row_v.at[slot], out_hbm.at[oitems + c],
                                  wsem.at[slot]).wait()

        def body(o, carry):
            for p in range(RING):
                c = RING * o + p

                @pl.when(c >= RING)
                def _(c=c, p=p):
                    wait_scatter(c - RING, p)

                fire_gather(c, p)

                c2 = c - DEFER
                slot2 = (p + DEFER) % RING

                @pl.when(c2 >= 0)
                def _(c2=c2, slot2=slot2):
                    wait_gather(c2, slot2)
                    fire_scatter(c2, slot2)

            return carry

        lax.fori_loop(0, NB, body, 0)

        for p in range(RING - DEFER, RING):
            c = NCHUNK - RING + p
            wait_gather(c, p)
            fire_scatter(c, p)
        for p in range(RING):
            wait_scatter(NCHUNK - RING + p, p)


@jax.jit
def _sc_gather(prefix_flat, table_r):
    mesh = plsc.VectorSubcoreMesh(core_axis_name="c", subcore_axis_name="s")
    f = pl.kernel(
        _sc_body,
        out_type=jax.ShapeDtypeStruct((NITEMS * NCHUNK, DC), jnp.float32),
        mesh=mesh,
        scratch_types=[
            pltpu.VMEM((ROWS_PER_W,), jnp.int32),
            pltpu.VMEM((RING, 16, DC), jnp.float32),
            pltpu.SemaphoreType.DMA((RING,)),
            pltpu.SemaphoreType.DMA((RING,)),
        ],
    )
    return f(prefix_flat, table_r)


def kernel(prefix, table):
    prefix_flat = prefix.reshape(NITEMS)
    table_r = table.reshape(PRE_SEQ_LEN * NCHUNK, DC)
    out_r = _sc_gather(prefix_flat, table_r)
    return out_r.reshape(B, PLEN, OUT_DIM)


# speedup vs baseline: 2.4695x; 2.2958x over previous
import jax
import jax.numpy as jnp
from jax import lax
from jax.experimental import pallas as pl
from jax.experimental.pallas import tpu as pltpu
from jax.experimental.pallas import tpu_sc as plsc

PRE_SEQ_LEN = 128
OUT_DIM = 49152
B = 8
PLEN = 128
NITEMS = B * PLEN
NCHUNK = 32
DC = OUT_DIM // NCHUNK
NC = 2
NS = 16
NW = NC * NS
ROWS_PER_W = NITEMS // NW
NG = ROWS_PER_W // 16
RING = 4
DEFER = RING // 2
NB = NCHUNK // RING


def _sc_body(prefix_hbm, table_hbm, out_hbm, pref_v, row_v, gsem, wsem):
    wid = lax.axis_index("s") * NC + lax.axis_index("c")
    base = wid * ROWS_PER_W
    pltpu.sync_copy(prefix_hbm.at[pl.ds(base, ROWS_PER_W)], pref_v)

    lanes = lax.iota(jnp.int32, 16)

    for h in range(NG):
        pv = pref_v[pl.ds(16 * h, 16)]
        oitems = base + 16 * h + lanes

        def fire_gather(c, slot):
            pltpu.async_copy(table_hbm.at[pv, pl.ds(c * DC, DC)],
                             row_v.at[slot], gsem.at[slot])

        def wait_gather(c, slot):
            pltpu.make_async_copy(table_hbm.at[pv, pl.ds(c * DC, DC)],
                                  row_v.at[slot], gsem.at[slot]).wait()

        def fire_scatter(c, slot):
            pltpu.async_copy(row_v.at[slot],
                             out_hbm.at[oitems, pl.ds(c * DC, DC)],
                             wsem.at[slot])

        def wait_scatter(c, slot):
            pltpu.make_async_copy(row_v.at[slot],
                                  out_hbm.at[oitems, pl.ds(c * DC, DC)],
                                  wsem.at[slot]).wait()

        def body(o, carry):
            for p in range(RING):
                c = RING * o + p

                @pl.when(c >= RING)
                def _(c=c, p=p):
                    wait_scatter(c - RING, p)

                fire_gather(c, p)

                c2 = c - DEFER
                slot2 = (p + DEFER) % RING

                @pl.when(c2 >= 0)
                def _(c2=c2, slot2=slot2):
                    wait_gather(c2, slot2)
                    fire_scatter(c2, slot2)

            return carry

        lax.fori_loop(0, NB, body, 0)

        for p in range(RING - DEFER, RING):
            c = NCHUNK - RING + p
            wait_gather(c, p)
            fire_scatter(c, p)
        for p in range(RING):
            wait_scatter(NCHUNK - RING + p, p)


@jax.jit
def _sc_gather(prefix_flat, table):
    mesh = plsc.VectorSubcoreMesh(core_axis_name="c", subcore_axis_name="s")
    f = pl.kernel(
        _sc_body,
        out_type=jax.ShapeDtypeStruct((NITEMS, OUT_DIM), jnp.float32),
        mesh=mesh,
        scratch_types=[
            pltpu.VMEM((ROWS_PER_W,), jnp.int32),
            pltpu.VMEM((RING, 16, DC), jnp.float32),
            pltpu.SemaphoreType.DMA((RING,)),
            pltpu.SemaphoreType.DMA((RING,)),
        ],
    )
    return f(prefix_flat, table)


def kernel(prefix, table):
    prefix_flat = prefix.reshape(NITEMS)
    out_r = _sc_gather(prefix_flat, table)
    return out_r.reshape(B, PLEN, OUT_DIM)


# speedup vs baseline: 2.5630x; 1.0379x over previous
import jax
import jax.numpy as jnp
from jax import lax
from jax.experimental import pallas as pl
from jax.experimental.pallas import tpu as pltpu
from jax.experimental.pallas import tpu_sc as plsc

PRE_SEQ_LEN = 128
OUT_DIM = 49152
B = 8
PLEN = 128
NITEMS = B * PLEN
NC = 2
NS = 16
NW = NC * NS
ROWS_PER_W = NITEMS // NW
IC = 2
DCI = OUT_DIM // IC
RING = 2 * IC
DEFER = RING // 2
STEPS = ROWS_PER_W * IC
NB = STEPS // RING


def _sc_body(prefix_hbm, table_hbm, out_hbm, pref_v, row_v, gsem, wsem):
    wid = lax.axis_index("s") * NC + lax.axis_index("c")
    base = wid * ROWS_PER_W
    pltpu.sync_copy(prefix_hbm.at[pl.ds(base, ROWS_PER_W), :], pref_v)

    def fire_gather(j, q, slot):
        pltpu.async_copy(
            table_hbm.at[pref_v.at[j], pl.ds(q * DCI, DCI)],
            row_v.at[slot], gsem.at[slot])

    def wait_gather(j, q, slot):
        pltpu.make_async_copy(
            table_hbm.at[pref_v.at[j], pl.ds(q * DCI, DCI)],
            row_v.at[slot], gsem.at[slot]).wait()

    def fire_scatter(j, q, slot):
        pltpu.async_copy(
            row_v.at[slot],
            out_hbm.at[pl.ds(base + j, 1), pl.ds(q * DCI, DCI)],
            wsem.at[slot])

    def wait_scatter(j, q, slot):
        pltpu.make_async_copy(
            row_v.at[slot],
            out_hbm.at[pl.ds(base + j, 1), pl.ds(q * DCI, DCI)],
            wsem.at[slot]).wait()

    def body(o, carry):
        for p in range(RING):
            s = RING * o + p
            j = 2 * o + p // IC
            q = p % IC

            @pl.when(s >= RING)
            def _(j=j, q=q, p=p):
                wait_scatter(j - 2, q, p)

            fire_gather(j, q, p)

            if p < IC:
                j2, q2, slot2 = 2 * o - 1, p, p + IC
            else:
                j2, q2, slot2 = 2 * o, p - IC, p - IC

            @pl.when(s - DEFER >= 0)
            def _(j2=j2, q2=q2, slot2=slot2):
                wait_gather(j2, q2, slot2)
                fire_scatter(j2, q2, slot2)

        return carry

    lax.fori_loop(0, NB, body, 0)

    for p in range(RING - DEFER, RING):
        s = STEPS - RING + p
        j, q = s // IC, p % IC
        wait_gather(j, q, p)
        fire_scatter(j, q, p)
    for p in range(RING):
        s = STEPS - RING + p
        j, q = s // IC, p % IC
        wait_scatter(j, q, p)


@jax.jit
def _sc_gather(prefix_col, table):
    mesh = plsc.VectorSubcoreMesh(core_axis_name="c", subcore_axis_name="s")
    f = pl.kernel(
        _sc_body,
        out_type=jax.ShapeDtypeStruct((NITEMS, OUT_DIM), jnp.float32),
        mesh=mesh,
        scratch_types=[
            pltpu.VMEM((ROWS_PER_W, 1), jnp.int32),
            pltpu.VMEM((RING, 1, DCI), jnp.float32),
            pltpu.SemaphoreType.DMA((RING,)),
            pltpu.SemaphoreType.DMA((RING,)),
        ],
    )
    return f(prefix_col, table)


def kernel(prefix, table):
    prefix_col = prefix.reshape(NITEMS, 1)
    out_r = _sc_gather(prefix_col, table)
    return out_r.reshape(B, PLEN, OUT_DIM)


# speedup vs baseline: 2.5712x; 1.0032x over previous
import jax
import jax.numpy as jnp
from jax import lax
from jax.experimental import pallas as pl
from jax.experimental.pallas import tpu as pltpu
from jax.experimental.pallas import tpu_sc as plsc

PRE_SEQ_LEN = 128
OUT_DIM = 49152
B = 8
PLEN = 128
NITEMS = B * PLEN
NC = 2
NS = 16
NW = NC * NS
ROWS_PER_W = NITEMS // NW
IC = 4
DCI = OUT_DIM // IC
RING = 2 * IC
DEFER = RING // 2
STEPS = ROWS_PER_W * IC
NB = STEPS // RING


def _sc_body(prefix_hbm, table_hbm, out_hbm, pref_v, row_v, gsem, wsem):
    wid = lax.axis_index("s") * NC + lax.axis_index("c")
    base = wid * ROWS_PER_W
    pltpu.sync_copy(prefix_hbm.at[pl.ds(base, ROWS_PER_W), :], pref_v)

    def fire_gather(j, q, slot):
        pltpu.async_copy(
            table_hbm.at[pref_v.at[j], pl.ds(q * DCI, DCI)],
            row_v.at[slot], gsem.at[slot])

    def wait_gather(j, q, slot):
        pltpu.make_async_copy(
            table_hbm.at[pref_v.at[j], pl.ds(q * DCI, DCI)],
            row_v.at[slot], gsem.at[slot]).wait()

    def fire_scatter(j, q, slot):
        pltpu.async_copy(
            row_v.at[slot],
            out_hbm.at[pl.ds(base + j, 1), pl.ds(q * DCI, DCI)],
            wsem.at[slot])

    def wait_scatter(j, q, slot):
        pltpu.make_async_copy(
            row_v.at[slot],
            out_hbm.at[pl.ds(base + j, 1), pl.ds(q * DCI, DCI)],
            wsem.at[slot]).wait()

    def body(o, carry):
        for p in range(RING):
            s = RING * o + p
            j = 2 * o + p // IC
            q = p % IC

            @pl.when(s >= RING)
            def _(j=j, q=q, p=p):
                wait_scatter(j - 2, q, p)

            fire_gather(j, q, p)

            if p < IC:
                j2, q2, slot2 = 2 * o - 1, p, p + IC
            else:
                j2, q2, slot2 = 2 * o, p - IC, p - IC

            @pl.when(s - DEFER >= 0)
            def _(j2=j2, q2=q2, slot2=slot2):
                wait_gather(j2, q2, slot2)
                fire_scatter(j2, q2, slot2)

        return carry

    lax.fori_loop(0, NB, body, 0)

    for p in range(RING - DEFER, RING):
        s = STEPS - RING + p
        j, q = s // IC, p % IC
        wait_gather(j, q, p)
        fire_scatter(j, q, p)
    for p in range(RING):
        s = STEPS - RING + p
        j, q = s // IC, p % IC
        wait_scatter(j, q, p)


@jax.jit
def _sc_gather(prefix_col, table):
    mesh = plsc.VectorSubcoreMesh(core_axis_name="c", subcore_axis_name="s")
    f = pl.kernel(
        _sc_body,
        out_type=jax.ShapeDtypeStruct((NITEMS, OUT_DIM), jnp.float32),
        mesh=mesh,
        scratch_types=[
            pltpu.VMEM((ROWS_PER_W, 1), jnp.int32),
            pltpu.VMEM((RING, 1, DCI), jnp.float32),
            pltpu.SemaphoreType.DMA((RING,)),
            pltpu.SemaphoreType.DMA((RING,)),
        ],
    )
    return f(prefix_col, table)


def kernel(prefix, table):
    prefix_col = prefix.reshape(NITEMS, 1)
    out_r = _sc_gather(prefix_col, table)
    return out_r.reshape(B, PLEN, OUT_DIM)
